# Initial kernel scaffold; baseline (speedup 1.0000x reference)
import jax
import jax.numpy as jnp
from jax.experimental import pallas as pl


def kernel(node_features, edge_features, senders, receivers, params):
    raise NotImplementedError("write your pallas kernel here")


# speedup vs baseline: 2.1587x; 2.1587x over previous
import functools

import jax
import jax.numpy as jnp
from jax import lax
from jax.experimental import pallas as pl
from jax.experimental.pallas import tpu as pltpu
from jax.experimental.pallas import tpu_sc as plsc

N_NODES = 10000
N_EDGES = 320000
D_FEAT = 128
LATENT = 64

NW = 32
EPW = N_EDGES // NW
CH = 80
NCH = EPW // CH
NPAD = 10240
HALF = NPAD // 2
RPS = HALF // 16
EPS2 = N_EDGES // 16
NCH2 = EPS2 // CH
EPS = 1e-5

BN = 2000
BE = 8000


def _collapse(p):
    (W, b) = p["layers"][0]
    for Wi, bi in p["layers"][1:]:
        W = W @ Wi
        b = b @ Wi + bi
    return W, b


def _ln(x, g, bt):
    mu = jnp.mean(x, axis=-1, keepdims=True)
    var = jnp.mean((x - mu) ** 2, axis=-1, keepdims=True)
    return (x - mu) / jnp.sqrt(var + EPS) * g + bt


def _wspec(shape):
    nd = len(shape)
    return pl.BlockSpec(shape, lambda i, _nd=nd: (0,) * _nd)


def _dot(a, b):
    return jnp.dot(a, b, preferred_element_type=jnp.float32,
                   precision=lax.Precision.HIGHEST)


def _enc_node_body(nf, Wen, ben, g, bt, Wsr, nl_o, T_o):
    x = _ln(_dot(nf[...], Wen[...]) + ben[...], g[...], bt[...])
    nl_o[...] = x
    T_o[...] = _dot(x, Wsr[...])


def _enc_edge_body(ef, Wee, bee, g, bt, We1, be1, el_o, E1_o):
    x = _ln(_dot(ef[...], Wee[...]) + bee[...], g[...], bt[...])
    el_o[...] = x
    E1_o[...] = _dot(x, We1[...]) + be1[...]


def _edge1_body(G, E, el, g, bt, We2, be2, ne_o, E2_o):
    ne = _ln(G[...] + E[...], g[...], bt[...])
    ne_o[...] = jnp.concatenate([ne, jnp.zeros_like(ne)], axis=-1)
    E2_o[...] = _dot(el[...] + ne, We2[...]) + be2[...]


def _edge2_body(G, E, g, bt, ne_o):
    ne = _ln(G[...] + E[...], g[...], bt[...])
    ne_o[...] = jnp.concatenate([ne, jnp.zeros_like(ne)], axis=-1)


def _node1_body(nl, agg_r, Wn, Wa, bn, g, bt, Wsr, nl_o, T_o):
    agg = agg_r[...][:, :_L]
    x = nl[...]
    nl2 = x + _ln(_dot(x, Wn[...]) + _dot(agg, Wa[...]) + bn[...],
                  g[...], bt[...])
    nl_o[...] = nl2
    T_o[...] = _dot(nl2, Wsr[...])


def _node2_body(nl, agg_r, Wn, Wa, bn, g, bt, Wd, bd, out_o):
    agg = agg_r[...][:, :_L]
    x = nl[...]
    nl2 = x + _ln(_dot(x, Wn[...]) + _dot(agg, Wa[...]) + bn[...],
                  g[...], bt[...])
    out_o[...] = _dot(nl2, Wd[...]) + bd[...]


def _row(blk, d):
    return pl.BlockSpec((blk, d), lambda i: (i, 0))


_f32 = jnp.float32
_L = LATENT


def _tc(body, grid, in_specs, out_shapes, out_specs):
    return pl.pallas_call(
        body,
        grid=(grid,),
        in_specs=in_specs,
        out_shape=[jax.ShapeDtypeStruct(s, _f32) for s in out_shapes],
        out_specs=out_specs,
    )


_SC_MESH = plsc.VectorSubcoreMesh(core_axis_name="c", subcore_axis_name="s")


@functools.partial(
    pl.kernel,
    mesh=_SC_MESH,
    out_type=jax.ShapeDtypeStruct((N_EDGES, _L), _f32),
    scratch_types=[
        pltpu.VMEM((NCH, CH), jnp.int32),
        pltpu.VMEM((NCH, CH), jnp.int32),
        pltpu.VMEM((CH, 2 * _L), _f32),
        pltpu.VMEM((CH, 2 * _L), _f32),
        pltpu.VMEM((CH, _L), _f32),
        pltpu.SemaphoreType.DMA,
        pltpu.SemaphoreType.DMA,
    ],
)
def _sc_gather(T_hbm, s3_hbm, r3_hbm, out_hbm,
               sidx, ridx, bufS, bufR, bufG, semA, semB):
    wid = lax.axis_index("s") * 2 + lax.axis_index("c")
    pltpu.sync_copy(s3_hbm.at[wid], sidx)
    pltpu.sync_copy(r3_hbm.at[wid], ridx)
    base = wid * EPW

    def chunk(i, carry):
        ca = pltpu.async_copy(T_hbm.at[sidx.at[i]], bufS, semA)
        cb = pltpu.async_copy(T_hbm.at[ridx.at[i]], bufR, semB)
        ca.wait()
        cb.wait()

        def row(j, c2):
            for k in range(4):
                sl = pl.ds(k * 16, 16)
                slr = pl.ds(_L + k * 16, 16)
                bufG[j, sl] = bufS[j, sl] + bufR[j, slr]
            return c2

        lax.fori_loop(0, CH, row, 0)
        pltpu.sync_copy(bufG, out_hbm.at[pl.ds(base + i * CH, CH)])
        return carry

    lax.fori_loop(0, NCH, chunk, 0)


@functools.partial(
    pl.kernel,
    mesh=_SC_MESH,
    out_type=jax.ShapeDtypeStruct((NPAD, 2 * _L), _f32),
    scratch_types=[
        pltpu.VMEM((NCH2, CH), jnp.int32),
        pltpu.VMEM((CH,), jnp.int32),
        pltpu.VMEM((CH, 2 * _L), _f32),
        pltpu.VMEM((CH, 2 * _L), _f32),
        pltpu.VMEM_SHARED((HALF + 16, 2 * _L), _f32),
        pltpu.SemaphoreType.DMA,
    ],
)
def _sc_scatter(ne_hbm, r16_hbm, out_hbm, ridx, cbuf, buf, zbuf,
                shared, sem):
    c = lax.axis_index("c")
    s = lax.axis_index("s")
    lo = c * HALF

    def zrow(j, carry):
        for k in range(8):
            zbuf[j, pl.ds(k * 16, 16)] = jnp.zeros((16,), _f32)
        return carry

    lax.fori_loop(0, CH, zrow, 0)

    def zt(t, carry):
        pltpu.sync_copy(zbuf, shared.at[pl.ds(s * RPS + t * CH, CH)])
        return carry

    lax.fori_loop(0, RPS // CH, zt, 0)
    plsc.subcore_barrier()

    pltpu.sync_copy(r16_hbm.at[s], ridx)
    base = s * EPS2

    def chunk(i, carry):
        pltpu.sync_copy(ne_hbm.at[pl.ds(base + i * CH, CH)], buf)
        for k in range(CH // 16):
            sl = pl.ds(k * 16, 16)
            vloc = ridx[i, sl] - lo
            ok = (vloc >= 0) & (vloc < HALF)
            cbuf[sl] = jnp.where(ok, vloc, HALF)
        pltpu.sync_copy(buf, shared.at[cbuf], add=True)
        return carry

    lax.fori_loop(0, NCH2, chunk, 0)
    plsc.subcore_barrier()

    def cp(t, carry):
        b = s * RPS + t * CH
        pltpu.sync_copy(shared.at[pl.ds(b, CH)], zbuf)
        pltpu.sync_copy(zbuf, out_hbm.at[pl.ds(lo + b, CH)])
        return carry

    lax.fori_loop(0, RPS // CH, cp, 0)


def kernel(node_features, edge_features, senders, receivers, params):
    Wen, ben = _collapse(params["enc_node"])
    gen, bten = params["enc_node"]["ln"]
    Wee, bee = _collapse(params["enc_edge"])
    gee, btee = params["enc_edge"]["ln"]

    blks = []
    for blk in params["blocks"]:
        We, be = _collapse(blk["edge"])
        ge, bte = blk["edge"]["ln"]
        Wn, bn = _collapse(blk["node"])
        gn, btn = blk["node"]["ln"]
        blks.append(dict(
            Wsr=jnp.concatenate([We[:_L], We[_L:2 * _L]], axis=1),
            We=We[2 * _L:], be=be,
            ge=ge, bte=bte, Wn=Wn[:_L], Wa=Wn[_L:], bn=bn, gn=gn, btn=btn))
    Wd, bd = _collapse(params["dec"])
    Wd8 = jnp.zeros((_L, 8), _f32).at[:, :3].set(Wd)
    bd8 = jnp.zeros((8,), _f32).at[:3].set(bd)

    def v(x):
        return x.reshape(1, -1)

    s3 = senders.reshape(NW, NCH, CH)
    r3 = receivers.reshape(NW, NCH, CH)
    r16 = receivers.reshape(16, NCH2, CH)

    b0, b1 = blks

    nl, T1 = _tc(
        _enc_node_body, N_NODES // BN,
        [_row(BN, D_FEAT), _wspec((D_FEAT, _L)), _wspec((1, _L)),
         _wspec((1, _L)), _wspec((1, _L)), _wspec((_L, 2 * _L))],
        [(N_NODES, _L), (N_NODES, 2 * _L)],
        [_row(BN, _L), _row(BN, 2 * _L)],
    )(node_features, Wen, v(ben), v(gen), v(bten), b0["Wsr"])

    el, E1 = _tc(
        _enc_edge_body, N_EDGES // BE,
        [_row(BE, 4), _wspec((4, _L)), _wspec((1, _L)), _wspec((1, _L)),
         _wspec((1, _L)), _wspec((_L, _L)), _wspec((1, _L))],
        [(N_EDGES, _L)] * 2,
        [_row(BE, _L)] * 2,
    )(edge_features, Wee, v(bee), v(gee), v(btee), b0["We"], v(b0["be"]))

    G1 = _sc_gather(T1, s3, r3)

    ne1, E2 = _tc(
        _edge1_body, N_EDGES // BE,
        [_row(BE, _L)] * 3 + [_wspec((1, _L)), _wspec((1, _L)),
                              _wspec((_L, _L)), _wspec((1, _L))],
        [(N_EDGES, 2 * _L), (N_EDGES, _L)],
        [_row(BE, 2 * _L), _row(BE, _L)],
    )(G1, E1, el, v(b0["ge"]), v(b0["bte"]), b1["We"], v(b1["be"]))

    agg1 = _sc_scatter(ne1, r16)

    nl2, T2 = _tc(
        _node1_body, N_NODES // BN,
        [_row(BN, _L), _row(BN, 2 * _L),
         _wspec((_L, _L)), _wspec((_L, _L)), _wspec((1, _L)),
         _wspec((1, _L)), _wspec((1, _L)), _wspec((_L, 2 * _L))],
        [(N_NODES, _L), (N_NODES, 2 * _L)],
        [_row(BN, _L), _row(BN, 2 * _L)],
    )(nl, agg1, b0["Wn"], b0["Wa"], v(b0["bn"]), v(b0["gn"]), v(b0["btn"]),
      b1["Wsr"])

    G2 = _sc_gather(T2, s3, r3)

    (ne2,) = _tc(
        _edge2_body, N_EDGES // BE,
        [_row(BE, _L)] * 2 + [_wspec((1, _L)), _wspec((1, _L))],
        [(N_EDGES, 2 * _L)],
        [_row(BE, 2 * _L)],
    )(G2, E2, v(b1["ge"]), v(b1["bte"]))

    agg2 = _sc_scatter(ne2, r16)

    (out8,) = _tc(
        _node2_body, N_NODES // BN,
        [_row(BN, _L), _row(BN, 2 * _L),
         _wspec((_L, _L)), _wspec((_L, _L)), _wspec((1, _L)),
         _wspec((1, _L)), _wspec((1, _L)), _wspec((_L, 8)), _wspec((1, 8))],
        [(N_NODES, 8)],
        [_row(BN, 8)],
    )(nl2, agg2, b1["Wn"], b1["Wa"], v(b1["bn"]), v(b1["gn"]), v(b1["btn"]),
      Wd8, v(bd8))

    return out8[:, :3]


# speedup vs baseline: 2.4372x; 1.1290x over previous
import functools

import jax
import jax.numpy as jnp
from jax import lax
from jax.experimental import pallas as pl
from jax.experimental.pallas import tpu as pltpu
from jax.experimental.pallas import tpu_sc as plsc

N_NODES = 10000
N_EDGES = 320000
D_FEAT = 128
LATENT = 64

NW = 32
EPW = N_EDGES // NW
CH = 80
NCH = EPW // CH
NPAD = 10240
HALF = NPAD // 2
RPS = HALF // 16
EPS2 = N_EDGES // 16
NCH2 = EPS2 // CH
EPS = 1e-5

BN = 2000
BE = 8000


def _collapse(p):
    (W, b) = p["layers"][0]
    for Wi, bi in p["layers"][1:]:
        W = W @ Wi
        b = b @ Wi + bi
    return W, b


def _ln(x, g, bt):
    mu = jnp.mean(x, axis=-1, keepdims=True)
    var = jnp.mean((x - mu) ** 2, axis=-1, keepdims=True)
    return (x - mu) / jnp.sqrt(var + EPS) * g + bt


def _wspec(shape):
    nd = len(shape)
    return pl.BlockSpec(shape, lambda i, _nd=nd: (0,) * _nd)


def _dot(a, b):
    return jnp.dot(a, b, preferred_element_type=jnp.float32,
                   precision=lax.Precision.HIGHEST)


def _enc_node_body(nf, Wen, ben, g, bt, Wsr, nl_o, T_o):
    x = _ln(_dot(nf[...], Wen[...]) + ben[...], g[...], bt[...])
    nl_o[...] = x
    T_o[...] = _dot(x, Wsr[...])


def _enc_edge_body(ef, Wee, bee, g, bt, We1, be1, el_o, E1_o):
    x = _ln(_dot(ef[...], Wee[...]) + bee[...], g[...], bt[...])
    el_o[...] = x
    E1_o[...] = _dot(x, We1[...]) + be1[...]


def _edge1_body(G, E, el, g, bt, We2, be2, ne_o, E2_o):
    ne = _ln(G[...] + E[...], g[...], bt[...])
    ne_o[...] = jnp.concatenate([ne, jnp.zeros_like(ne)], axis=-1)
    E2_o[...] = _dot(el[...] + ne, We2[...]) + be2[...]


def _edge2_body(G, E, g, bt, ne_o):
    ne = _ln(G[...] + E[...], g[...], bt[...])
    ne_o[...] = jnp.concatenate([ne, jnp.zeros_like(ne)], axis=-1)


def _node1_body(nl, agg_r, Wn, Wa, bn, g, bt, Wsr, nl_o, T_o):
    agg = agg_r[...][:, :_L]
    x = nl[...]
    nl2 = x + _ln(_dot(x, Wn[...]) + _dot(agg, Wa[...]) + bn[...],
                  g[...], bt[...])
    nl_o[...] = nl2
    T_o[...] = _dot(nl2, Wsr[...])


def _node2_body(nl, agg_r, Wn, Wa, bn, g, bt, Wd, bd, out_o):
    agg = agg_r[...][:, :_L]
    x = nl[...]
    nl2 = x + _ln(_dot(x, Wn[...]) + _dot(agg, Wa[...]) + bn[...],
                  g[...], bt[...])
    out_o[...] = _dot(nl2, Wd[...]) + bd[...]


def _row(blk, d):
    return pl.BlockSpec((blk, d), lambda i: (i, 0))


_f32 = jnp.float32
_L = LATENT


def _tc(body, grid, in_specs, out_shapes, out_specs):
    return pl.pallas_call(
        body,
        grid=(grid,),
        in_specs=in_specs,
        out_shape=[jax.ShapeDtypeStruct(s, _f32) for s in out_shapes],
        out_specs=out_specs,
    )


_SC_MESH = plsc.VectorSubcoreMesh(core_axis_name="c", subcore_axis_name="s")


@functools.partial(
    pl.kernel,
    mesh=_SC_MESH,
    out_type=jax.ShapeDtypeStruct((N_EDGES, _L), _f32),
    scratch_types=[
        pltpu.VMEM((NCH, CH), jnp.int32),
        pltpu.VMEM((NCH, CH), jnp.int32),
        pltpu.VMEM((CH, 2 * _L), _f32),
        pltpu.VMEM((CH, 2 * _L), _f32),
        pltpu.VMEM((CH, 2 * _L), _f32),
        pltpu.VMEM((CH, 2 * _L), _f32),
        pltpu.VMEM((CH, _L), _f32),
        pltpu.SemaphoreType.DMA,
        pltpu.SemaphoreType.DMA,
        pltpu.SemaphoreType.DMA,
        pltpu.SemaphoreType.DMA,
    ],
)
def _sc_gather(T_hbm, s3_hbm, r3_hbm, out_hbm, sidx, ridx,
               bufS0, bufR0, bufS1, bufR1, bufG, sS0, sR0, sS1, sR1):
    wid = lax.axis_index("s") * 2 + lax.axis_index("c")
    pltpu.sync_copy(s3_hbm.at[wid], sidx)
    pltpu.sync_copy(r3_hbm.at[wid], ridx)
    base = wid * EPW

    def start(row, bS, bR, s1, s2):
        pltpu.async_copy(T_hbm.at[sidx.at[row]], bS, s1)
        pltpu.async_copy(T_hbm.at[ridx.at[row]], bR, s2)

    def wait(row, bS, bR, s1, s2):
        pltpu.make_async_copy(T_hbm.at[sidx.at[row]], bS, s1).wait()
        pltpu.make_async_copy(T_hbm.at[ridx.at[row]], bR, s2).wait()

    def addwrite(row, bS, bR):
        def rowfn(j, c2):
            for k in range(4):
                bufG[j, pl.ds(k * 16, 16)] = (bS[j, pl.ds(k * 16, 16)]
                                              + bR[j, pl.ds(_L + k * 16, 16)])
            return c2

        lax.fori_loop(0, CH, rowfn, 0)
        pltpu.sync_copy(bufG, out_hbm.at[pl.ds(base + row * CH, CH)])

    start(0, bufS0, bufR0, sS0, sR0)

    def body(t, carry):
        i0 = 2 * t
        wait(i0, bufS0, bufR0, sS0, sR0)
        start(i0 + 1, bufS1, bufR1, sS1, sR1)
        addwrite(i0, bufS0, bufR0)
        wait(i0 + 1, bufS1, bufR1, sS1, sR1)
        start(i0 + 2, bufS0, bufR0, sS0, sR0)
        addwrite(i0 + 1, bufS1, bufR1)
        return carry

    lax.fori_loop(0, NCH // 2, body, 0)
    wait(NCH - 1, bufS0, bufR0, sS0, sR0)
    addwrite(NCH - 1, bufS0, bufR0)


@functools.partial(
    pl.kernel,
    mesh=_SC_MESH,
    out_type=jax.ShapeDtypeStruct((NPAD, 2 * _L), _f32),
    scratch_types=[
        pltpu.VMEM((NCH2, CH), jnp.int32),
        pltpu.VMEM((CH,), jnp.int32),
        pltpu.VMEM((CH,), jnp.int32),
        pltpu.VMEM((CH, 2 * _L), _f32),
        pltpu.VMEM((CH, 2 * _L), _f32),
        pltpu.VMEM((CH, 2 * _L), _f32),
        pltpu.VMEM_SHARED((HALF + 16, 2 * _L), _f32),
        pltpu.SemaphoreType.DMA,
        pltpu.SemaphoreType.DMA,
        pltpu.SemaphoreType.DMA,
        pltpu.SemaphoreType.DMA,
    ],
)
def _sc_scatter(ne_hbm, r16_hbm, out_hbm, ridx, cb0, cb1, b0, b1, zbuf,
                shared, sRd0, sRd1, sAd0, sAd1):
    c = lax.axis_index("c")
    s = lax.axis_index("s")
    lo = c * HALF

    def zrow(j, carry):
        for k in range(8):
            zbuf[j, pl.ds(k * 16, 16)] = jnp.zeros((16,), _f32)
        return carry

    lax.fori_loop(0, CH, zrow, 0)

    def zt(t, carry):
        pltpu.sync_copy(zbuf, shared.at[pl.ds(s * RPS + t * CH, CH)])
        return carry

    lax.fori_loop(0, RPS // CH, zt, 0)
    pltpu.sync_copy(r16_hbm.at[s], ridx)
    plsc.subcore_barrier()

    base = s * EPS2

    def startrd(row, b, sm):
        pltpu.async_copy(ne_hbm.at[pl.ds(base + row * CH, CH)], b, sm)

    def waitrd(row, b, sm):
        pltpu.make_async_copy(ne_hbm.at[pl.ds(base + row * CH, CH)],
                              b, sm).wait()

    def mkcb(row, cb):
        for k in range(CH // 16):
            sl = pl.ds(k * 16, 16)
            vloc = ridx[row, sl] - lo
            ok = (vloc >= 0) & (vloc < HALF)
            cb[sl] = jnp.where(ok, vloc, HALF)

    def startadd(b, cb, sm):
        pltpu.make_async_copy(b, shared.at[cb], sm).start(add=True)

    def waitadd(b, cb, sm):
        pltpu.make_async_copy(b, shared.at[cb], sm).wait()

    startrd(0, b0, sRd0)

    def body(t, carry):
        i0 = 2 * t
        waitrd(i0, b0, sRd0)
        startrd(i0 + 1, b1, sRd1)
        mkcb(i0, cb0)
        startadd(b0, cb0, sAd0)
        waitrd(i0 + 1, b1, sRd1)
        mkcb(i0 + 1, cb1)
        startadd(b1, cb1, sAd1)
        waitadd(b0, cb0, sAd0)
        startrd(i0 + 2, b0, sRd0)
        waitadd(b1, cb1, sAd1)
        return carry

    lax.fori_loop(0, NCH2 // 2 - 1, body, 0)
    i0 = NCH2 - 2
    waitrd(i0, b0, sRd0)
    startrd(i0 + 1, b1, sRd1)
    mkcb(i0, cb0)
    startadd(b0, cb0, sAd0)
    waitrd(i0 + 1, b1, sRd1)
    mkcb(i0 + 1, cb1)
    startadd(b1, cb1, sAd1)
    waitadd(b0, cb0, sAd0)
    waitadd(b1, cb1, sAd1)
    plsc.subcore_barrier()

    def cp(t, carry):
        b = s * RPS + t * CH
        pltpu.sync_copy(shared.at[pl.ds(b, CH)], zbuf)
        pltpu.sync_copy(zbuf, out_hbm.at[pl.ds(lo + b, CH)])
        return carry

    lax.fori_loop(0, RPS // CH, cp, 0)


def kernel(node_features, edge_features, senders, receivers, params):
    Wen, ben = _collapse(params["enc_node"])
    gen, bten = params["enc_node"]["ln"]
    Wee, bee = _collapse(params["enc_edge"])
    gee, btee = params["enc_edge"]["ln"]

    blks = []
    for blk in params["blocks"]:
        We, be = _collapse(blk["edge"])
        ge, bte = blk["edge"]["ln"]
        Wn, bn = _collapse(blk["node"])
        gn, btn = blk["node"]["ln"]
        blks.append(dict(
            Wsr=jnp.concatenate([We[:_L], We[_L:2 * _L]], axis=1),
            We=We[2 * _L:], be=be,
            ge=ge, bte=bte, Wn=Wn[:_L], Wa=Wn[_L:], bn=bn, gn=gn, btn=btn))
    Wd, bd = _collapse(params["dec"])
    Wd8 = jnp.zeros((_L, 8), _f32).at[:, :3].set(Wd)
    bd8 = jnp.zeros((8,), _f32).at[:3].set(bd)

    def v(x):
        return x.reshape(1, -1)

    s3 = senders.reshape(NW, NCH, CH)
    r3 = receivers.reshape(NW, NCH, CH)
    r16 = receivers.reshape(16, NCH2, CH)

    b0, b1 = blks

    nl, T1 = _tc(
        _enc_node_body, N_NODES // BN,
        [_row(BN, D_FEAT), _wspec((D_FEAT, _L)), _wspec((1, _L)),
         _wspec((1, _L)), _wspec((1, _L)), _wspec((_L, 2 * _L))],
        [(N_NODES, _L), (N_NODES, 2 * _L)],
        [_row(BN, _L), _row(BN, 2 * _L)],
    )(node_features, Wen, v(ben), v(gen), v(bten), b0["Wsr"])

    el, E1 = _tc(
        _enc_edge_body, N_EDGES // BE,
        [_row(BE, 4), _wspec((4, _L)), _wspec((1, _L)), _wspec((1, _L)),
         _wspec((1, _L)), _wspec((_L, _L)), _wspec((1, _L))],
        [(N_EDGES, _L)] * 2,
        [_row(BE, _L)] * 2,
    )(edge_features, Wee, v(bee), v(gee), v(btee), b0["We"], v(b0["be"]))

    G1 = _sc_gather(T1, s3, r3)

    ne1, E2 = _tc(
        _edge1_body, N_EDGES // BE,
        [_row(BE, _L)] * 3 + [_wspec((1, _L)), _wspec((1, _L)),
                              _wspec((_L, _L)), _wspec((1, _L))],
        [(N_EDGES, 2 * _L), (N_EDGES, _L)],
        [_row(BE, 2 * _L), _row(BE, _L)],
    )(G1, E1, el, v(b0["ge"]), v(b0["bte"]), b1["We"], v(b1["be"]))

    agg1 = _sc_scatter(ne1, r16)

    nl2, T2 = _tc(
        _node1_body, N_NODES // BN,
        [_row(BN, _L), _row(BN, 2 * _L),
         _wspec((_L, _L)), _wspec((_L, _L)), _wspec((1, _L)),
         _wspec((1, _L)), _wspec((1, _L)), _wspec((_L, 2 * _L))],
        [(N_NODES, _L), (N_NODES, 2 * _L)],
        [_row(BN, _L), _row(BN, 2 * _L)],
    )(nl, agg1, b0["Wn"], b0["Wa"], v(b0["bn"]), v(b0["gn"]), v(b0["btn"]),
      b1["Wsr"])

    G2 = _sc_gather(T2, s3, r3)

    (ne2,) = _tc(
        _edge2_body, N_EDGES // BE,
        [_row(BE, _L)] * 2 + [_wspec((1, _L)), _wspec((1, _L))],
        [(N_EDGES, 2 * _L)],
        [_row(BE, 2 * _L)],
    )(G2, E2, v(b1["ge"]), v(b1["bte"]))

    agg2 = _sc_scatter(ne2, r16)

    (out8,) = _tc(
        _node2_body, N_NODES // BN,
        [_row(BN, _L), _row(BN, 2 * _L),
         _wspec((_L, _L)), _wspec((_L, _L)), _wspec((1, _L)),
         _wspec((1, _L)), _wspec((1, _L)), _wspec((_L, 8)), _wspec((1, 8))],
        [(N_NODES, 8)],
        [_row(BN, 8)],
    )(nl2, agg2, b1["Wn"], b1["Wa"], v(b1["bn"]), v(b1["gn"]), v(b1["btn"]),
      Wd8, v(bd8))

    return out8[:, :3]
